# baseline (speedup 1.0000x reference)
import functools

import numpy as np
import jax
import jax.numpy as jnp
from jax import lax
from jax.experimental import pallas as pl
from jax.experimental.pallas import tpu as pltpu
from jax.experimental.pallas import tpu_sc as plsc

L_LEVELS = 8
T_TBL = 2 ** 19
TMASK = T_TBL - 1
TRES, XRES, YRES = 32, 64, 64
C3 = 36
C3P = 48
EMB = 84
HID = 64
P2_I32 = np.int32(np.uint32(2654435761).view(np.int32))

NC, NS = 2, 16
NW = NC * NS
B = 128

_OFF3 = [dt * (XRES * YRES) + dx * YRES + dy
         for dt in (0, 1) for dx in (0, 1) for dy in (0, 1)]
_PLANES = [(1, 2), (0, 2), (0, 1)]


def _emb_body(nblk, ac0, ac1, ac2, t0, t1, t2, grid, emb_out,
              xyz_v, idx_v, w_flat, rows_flat, idx3_v, w3f, rows3_v,
              dsp_v, sblock_v, sem):
    wid = lax.axis_index("s") * NC + lax.axis_index("c")
    iota = lax.iota(jnp.int32, 16)
    tables = [t0, t1, t2]

    def _dsp(g, c):
        q = g * 16 + iota
        dsp_v[pl.ds(g * 16, 16)] = q * EMB
        return c
    lax.fori_loop(0, B // 16, _dsp, 0)

    def block(bi, carry):
        base = (wid * nblk + bi) * B

        for d, ac in enumerate((ac0, ac1, ac2)):
            pltpu.sync_copy(ac.at[pl.ds(base, B)], xyz_v.at[d])

        def mgrp(g, c):
            s = g * 16
            ct = xyz_v[0, pl.ds(s, 16)]
            cx = xyz_v[1, pl.ds(s, 16)]
            cy = xyz_v[2, pl.ds(s, 16)]
            pt = ct * jnp.float32(TRES - 1)
            px = cx * jnp.float32(XRES - 1)
            py = cy * jnp.float32(YRES - 1)
            ft = jnp.clip(pt.astype(jnp.int32), 0, TRES - 2)
            fx = jnp.clip(px.astype(jnp.int32), 0, XRES - 2)
            fy = jnp.clip(py.astype(jnp.int32), 0, YRES - 2)
            wt = pt - ft.astype(jnp.float32)
            wx = px - fx.astype(jnp.float32)
            wy = py - fy.astype(jnp.float32)
            fl = ft * (XRES * YRES) + fx * YRES + fy
            wts = (1.0 - wt, wt)
            wxs = (1.0 - wx, wx)
            wys = (1.0 - wy, wy)
            ci = 0
            for dt in (0, 1):
                for dx in (0, 1):
                    wtx = wts[dt] * wxs[dx]
                    for dy in (0, 1):
                        idx3_v[ci, pl.ds(s, 16)] = fl + _OFF3[ci]
                        w3f[pl.ds(ci * B + s, 16)] = wtx * wys[dy]
                        ci += 1
            return c
        lax.fori_loop(0, B // 16, mgrp, 0)

        cps = [pltpu.async_copy(grid.at[idx3_v.at[ci]], rows3_v.at[ci], sem)
               for ci in range(8)]
        for cp in cps:
            cp.wait()

        def macc(p, c):
            ws = [plsc.load_gather(w3f, [jnp.full((16,), ci * B + p, jnp.int32)])
                  for ci in range(8)]
            for gi in range(3):
                acc = ws[0] * rows3_v[0, p, pl.ds(gi * 16, 16)]
                for ci in range(1, 8):
                    acc = acc + ws[ci] * rows3_v[ci, p, pl.ds(gi * 16, 16)]
                sblock_v[pl.ds(p * EMB + 48 + gi * 16, 16)] = acc
            return c
        lax.fori_loop(0, B, macc, 0)

        for p, (d0, d1) in enumerate(_PLANES):
            def lvl(l, rf, _p=p, _d0=d0, _d1=d1):
                resf = rf.astype(jnp.int32).astype(jnp.float32)
                off = l * T_TBL

                def hgrp(g, c):
                    s = g * 16
                    u = xyz_v[_d0, pl.ds(s, 16)]
                    v = xyz_v[_d1, pl.ds(s, 16)]
                    pu = u * resf
                    pv = v * resf
                    fu = pu.astype(jnp.int32)
                    fv = pv.astype(jnp.int32)
                    wu = pu - fu.astype(jnp.float32)
                    wv = pv - fv.astype(jnp.float32)
                    hv0 = fv * P2_I32
                    hv1 = (fv + 1) * P2_I32
                    wus = (1.0 - wu, wu)
                    wvs = (1.0 - wv, wv)
                    ci = 0
                    for du in (0, 1):
                        cu = fu + du
                        for dv in (0, 1):
                            hh = cu ^ (hv1 if dv else hv0)
                            ie = ((hh & TMASK) + off) * 2
                            idx_v[2 * ci, pl.ds(s, 16)] = ie
                            idx_v[2 * ci + 1, pl.ds(s, 16)] = ie + 1
                            w_flat[pl.ds(ci * B + s, 16)] = wus[du] * wvs[dv]
                            ci += 1
                    return c
                lax.fori_loop(0, B // 16, hgrp, 0)

                gcps = [pltpu.async_copy(tables[_p].at[idx_v.at[k]],
                                         rows_flat.at[pl.ds(k * B, B)],
                                         sem)
                        for k in range(8)]
                for cp in gcps:
                    cp.wait()

                colbase = _p * 16 + l * 2

                def agrp(g, c):
                    s = g * 16
                    dd = dsp_v[pl.ds(s, 16)] + colbase
                    acc0 = None
                    acc1 = None
                    for ci in range(4):
                        wgt = w_flat[pl.ds(ci * B + s, 16)]
                        r0 = rows_flat[pl.ds(2 * ci * B + s, 16)]
                        r1 = rows_flat[pl.ds((2 * ci + 1) * B + s, 16)]
                        acc0 = wgt * r0 if acc0 is None else acc0 + wgt * r0
                        acc1 = wgt * r1 if acc1 is None else acc1 + wgt * r1
                    plsc.store_scatter(sblock_v, [dd], acc0)
                    plsc.store_scatter(sblock_v, [dd + 1], acc1)
                    return c
                lax.fori_loop(0, B // 16, agrp, 0)
                return rf * 1.5

            lax.fori_loop(0, L_LEVELS, lvl, jnp.full((16,), 16.0, jnp.float32))

        pltpu.sync_copy(sblock_v.at[pl.ds(0, B * EMB)],
                        emb_out.at[pl.ds(base * EMB, B * EMB)])
        return carry

    lax.fori_loop(0, nblk, block, 0)


def _sc_embed(ac0, ac1, ac2, t0, t1, t2, grid, n):
    nblk = n // (NW * B)
    mesh = plsc.VectorSubcoreMesh(core_axis_name="c", subcore_axis_name="s")
    return pl.kernel(
        functools.partial(_emb_body, nblk),
        out_type=jax.ShapeDtypeStruct((n * EMB,), jnp.float32),
        mesh=mesh,
        compiler_params=pltpu.CompilerParams(needs_layout_passes=False,
                                             use_tc_tiling_on_sc=False),
        scratch_types=[
            pltpu.VMEM((3, B), jnp.float32),
            pltpu.VMEM((8, B), jnp.int32),
            pltpu.VMEM((4 * B,), jnp.float32),
            pltpu.VMEM((8 * B,), jnp.float32),
            pltpu.VMEM((8, B), jnp.int32),
            pltpu.VMEM((8 * B,), jnp.float32),
            pltpu.VMEM((8, B, C3P), jnp.float32),
            pltpu.VMEM((B,), jnp.int32),
            pltpu.VMEM((B * EMB + 16,), jnp.float32),
            pltpu.SemaphoreType.DMA,
        ],
    )(ac0, ac1, ac2, t0, t1, t2, grid)


def _mlp_body(ts_ref, emb_ref, m0, mb0, m1a, m1b, mb1, m2a, m2b, mb2,
              w0, b0, w1, b1, w2, b2, wl, bl, out_ref):
    dot = functools.partial(jnp.dot, precision=lax.Precision.HIGHEST)
    emb = emb_ref[...]
    h0 = jnp.maximum(dot(emb, m0[...]) + mb0[...], 0.0)
    h1 = jnp.maximum(dot(h0, m1a[...]) + dot(emb, m1b[...]) + mb1[...], 0.0)
    h2 = jnp.maximum(dot(h1, m2a[...]) + dot(emb, m2b[...]) + mb2[...], 0.0)
    x = jnp.sin(30.0 * (ts_ref[...] * w0[...] + b0[...])) * h0
    x = jnp.sin(dot(x, w1[...]) + b1[...]) * h1
    x = jnp.sin(dot(x, w2[...]) + b2[...]) * h2
    out_ref[...] = dot(x, wl[...]) + bl[...]


def _mlp(ts, emb, weights, n):
    R = 4096
    grid = n // R
    full = lambda w: pl.BlockSpec(w.shape, lambda i: (0,) * w.ndim)
    return pl.pallas_call(
        _mlp_body,
        grid=(grid,),
        in_specs=[pl.BlockSpec((R, 1), lambda i: (i, 0)),
                  pl.BlockSpec((R, EMB), lambda i: (i, 0))]
                 + [full(w) for w in weights],
        out_specs=pl.BlockSpec((R, 3), lambda i: (i, 0)),
        out_shape=jax.ShapeDtypeStruct((n, 3), jnp.float32),
    )(ts, emb, *weights)


def kernel(temporal_steps, all_coords, table_xy, table_yt, table_xt, grid3d,
           W0, b0, W1, b1, W2, b2, Wl, bl, M0, Mb0, M1, Mb1, M2, Mb2):
    b, t = temporal_steps.shape
    n = b * t
    ts = temporal_steps.reshape(n, 1)
    ac = all_coords.reshape(n, 3)
    ac0, ac1, ac2 = ac[:, 0], ac[:, 1], ac[:, 2]

    t0 = table_xy.reshape(-1)
    t1 = table_yt.reshape(-1)
    t2 = table_xt.reshape(-1)
    grid = jnp.pad(grid3d, ((0, 0), (0, 0), (0, 0), (0, C3P - C3))
                   ).reshape(TRES * XRES * YRES, C3P)

    emb = _sc_embed(ac0, ac1, ac2, t0, t1, t2, grid, n).reshape(n, EMB)

    weights = (
        M0.T, Mb0.reshape(1, HID),
        M1[:, :HID].T, M1[:, HID:].T, Mb1.reshape(1, HID),
        M2[:, :HID].T, M2[:, HID:].T, Mb2.reshape(1, HID),
        W0.T, b0.reshape(1, HID),
        W1.T, b1.reshape(1, HID),
        W2.T, b2.reshape(1, HID),
        Wl.T, bl.reshape(1, 3),
    )
    out = _mlp(ts, emb, weights, n)
    return out.reshape(b, t, 3)


# speedup vs baseline: 1.0274x; 1.0274x over previous
import functools

import numpy as np
import jax
import jax.numpy as jnp
from jax import lax
from jax.experimental import pallas as pl
from jax.experimental.pallas import tpu as pltpu
from jax.experimental.pallas import tpu_sc as plsc

L_LEVELS = 8
T_TBL = 2 ** 19
TMASK = T_TBL - 1
TRES, XRES, YRES = 32, 64, 64
C3 = 36
C3P = 48
EMB = 84
HID = 64
P2_I32 = np.int32(np.uint32(2654435761).view(np.int32))

NC, NS = 2, 16
NW = NC * NS
B = 128

_OFF3 = [dt * (XRES * YRES) + dx * YRES + dy
         for dt in (0, 1) for dx in (0, 1) for dy in (0, 1)]
_PLANES = [(1, 2), (0, 2), (0, 1)]


def _emb_body(nblk, ac0, ac1, ac2, t0, t1, t2, grid, emb_out,
              xyz_v, idx_all, w_all, rows_all, idx3_v, w3f, rows3_v,
              dsp_v, sblock_v, sem_m, sem_s):
    wid = lax.axis_index("s") * NC + lax.axis_index("c")
    iota = lax.iota(jnp.int32, 16)
    tables = [t0, t1, t2]

    def _dsp(g, c):
        q = g * 16 + iota
        dsp_v[pl.ds(g * 16, 16)] = q * EMB
        return c
    lax.fori_loop(0, B // 16, _dsp, 0)

    def block(bi, carry):
        base = (wid * nblk + bi) * B

        for d, ac in enumerate((ac0, ac1, ac2)):
            pltpu.sync_copy(ac.at[pl.ds(base, B)], xyz_v.at[d])

        def mgrp(g, c):
            s = g * 16
            ct = xyz_v[0, pl.ds(s, 16)]
            cx = xyz_v[1, pl.ds(s, 16)]
            cy = xyz_v[2, pl.ds(s, 16)]
            pt = ct * jnp.float32(TRES - 1)
            px = cx * jnp.float32(XRES - 1)
            py = cy * jnp.float32(YRES - 1)
            ft = jnp.clip(pt.astype(jnp.int32), 0, TRES - 2)
            fx = jnp.clip(px.astype(jnp.int32), 0, XRES - 2)
            fy = jnp.clip(py.astype(jnp.int32), 0, YRES - 2)
            wt = pt - ft.astype(jnp.float32)
            wx = px - fx.astype(jnp.float32)
            wy = py - fy.astype(jnp.float32)
            fl = ft * (XRES * YRES) + fx * YRES + fy
            wts = (1.0 - wt, wt)
            wxs = (1.0 - wx, wx)
            wys = (1.0 - wy, wy)
            ci = 0
            for dt in (0, 1):
                for dx in (0, 1):
                    wtx = wts[dt] * wxs[dx]
                    for dy in (0, 1):
                        idx3_v[ci, pl.ds(s, 16)] = fl + _OFF3[ci]
                        w3f[pl.ds(ci * B + s, 16)] = wtx * wys[dy]
                        ci += 1
            return c
        lax.fori_loop(0, B // 16, mgrp, 0)

        mcps = [pltpu.async_copy(grid.at[idx3_v.at[ci]], rows3_v.at[ci],
                                 sem_m)
                for ci in range(8)]

        for p, (d0, d1) in enumerate(_PLANES):
            def lvl(l, rf, _p=p, _d0=d0, _d1=d1):
                resf = rf.astype(jnp.int32).astype(jnp.float32)
                off = l * T_TBL
                slot = _p * L_LEVELS + l

                def hgrp(g, c):
                    s = g * 16
                    u = xyz_v[_d0, pl.ds(s, 16)]
                    v = xyz_v[_d1, pl.ds(s, 16)]
                    pu = u * resf
                    pv = v * resf
                    fu = pu.astype(jnp.int32)
                    fv = pv.astype(jnp.int32)
                    wu = pu - fu.astype(jnp.float32)
                    wv = pv - fv.astype(jnp.float32)
                    hv0 = fv * P2_I32
                    hv1 = (fv + 1) * P2_I32
                    wus = (1.0 - wu, wu)
                    wvs = (1.0 - wv, wv)
                    ci = 0
                    for du in (0, 1):
                        cu = fu + du
                        for dv in (0, 1):
                            hh = cu ^ (hv1 if dv else hv0)
                            ie = ((hh & TMASK) + off) * 2
                            idx_all[slot, 2 * ci, pl.ds(s, 16)] = ie
                            idx_all[slot, 2 * ci + 1, pl.ds(s, 16)] = ie + 1
                            w_all[slot, pl.ds(ci * B + s, 16)] = \
                                wus[du] * wvs[dv]
                            ci += 1
                    return c
                lax.fori_loop(0, B // 16, hgrp, 0)

                for k in range(8):
                    pltpu.async_copy(tables[_p].at[idx_all.at[slot, k]],
                                     rows_all.at[slot, pl.ds(k * B, B)],
                                     sem_s)
                return rf * 1.5

            lax.fori_loop(0, L_LEVELS, lvl, jnp.full((16,), 16.0, jnp.float32))

        for cp in mcps:
            cp.wait()

        def macc(p, c):
            ws = [plsc.load_gather(w3f, [jnp.full((16,), ci * B + p, jnp.int32)])
                  for ci in range(8)]
            for gi in range(3):
                acc = ws[0] * rows3_v[0, p, pl.ds(gi * 16, 16)]
                for ci in range(1, 8):
                    acc = acc + ws[ci] * rows3_v[ci, p, pl.ds(gi * 16, 16)]
                sblock_v[pl.ds(p * EMB + 48 + gi * 16, 16)] = acc
            return c
        lax.fori_loop(0, B, macc, 0)

        def sdrain(l, c):
            pltpu.make_async_copy(t0.at[pl.ds(0, 8 * B)], rows_all.at[l],
                                  sem_s).wait()
            return c
        lax.fori_loop(0, 3 * L_LEVELS, sdrain, 0)

        def sacc(l, c):
            p_ = l // L_LEVELS
            colbase = p_ * 16 + (l - p_ * L_LEVELS) * 2

            def agrp(g, cc):
                s = g * 16
                dd = dsp_v[pl.ds(s, 16)] + colbase
                acc0 = None
                acc1 = None
                for ci in range(4):
                    wgt = w_all[l, pl.ds(ci * B + s, 16)]
                    r0 = rows_all[l, pl.ds(2 * ci * B + s, 16)]
                    r1 = rows_all[l, pl.ds((2 * ci + 1) * B + s, 16)]
                    acc0 = wgt * r0 if acc0 is None else acc0 + wgt * r0
                    acc1 = wgt * r1 if acc1 is None else acc1 + wgt * r1
                plsc.store_scatter(sblock_v, [dd], acc0)
                plsc.store_scatter(sblock_v, [dd + 1], acc1)
                return cc
            lax.fori_loop(0, B // 16, agrp, 0)
            return c
        lax.fori_loop(0, 3 * L_LEVELS, sacc, 0)

        pltpu.sync_copy(sblock_v.at[pl.ds(0, B * EMB)],
                        emb_out.at[pl.ds(base * EMB, B * EMB)])
        return carry

    lax.fori_loop(0, nblk, block, 0)


def _sc_embed(ac0, ac1, ac2, t0, t1, t2, grid, n):
    nblk = n // (NW * B)
    mesh = plsc.VectorSubcoreMesh(core_axis_name="c", subcore_axis_name="s")
    return pl.kernel(
        functools.partial(_emb_body, nblk),
        out_type=jax.ShapeDtypeStruct((n * EMB,), jnp.float32),
        mesh=mesh,
        compiler_params=pltpu.CompilerParams(needs_layout_passes=False,
                                             use_tc_tiling_on_sc=False),
        scratch_types=[
            pltpu.VMEM((3, B), jnp.float32),
            pltpu.VMEM((24, 8, B), jnp.int32),
            pltpu.VMEM((24, 4 * B), jnp.float32),
            pltpu.VMEM((24, 8 * B), jnp.float32),
            pltpu.VMEM((8, B), jnp.int32),
            pltpu.VMEM((8 * B,), jnp.float32),
            pltpu.VMEM((8, B, C3P), jnp.float32),
            pltpu.VMEM((B,), jnp.int32),
            pltpu.VMEM((B * EMB + 16,), jnp.float32),
            pltpu.SemaphoreType.DMA,
            pltpu.SemaphoreType.DMA,
        ],
    )(ac0, ac1, ac2, t0, t1, t2, grid)


def _mlp_body(ts_ref, emb_ref, m0, mb0, m1a, m1b, mb1, m2a, m2b, mb2,
              w0, b0, w1, b1, w2, b2, wl, bl, out_ref):
    dot = functools.partial(jnp.dot, precision=lax.Precision.HIGHEST)
    emb = emb_ref[...]
    h0 = jnp.maximum(dot(emb, m0[...]) + mb0[...], 0.0)
    h1 = jnp.maximum(dot(h0, m1a[...]) + dot(emb, m1b[...]) + mb1[...], 0.0)
    h2 = jnp.maximum(dot(h1, m2a[...]) + dot(emb, m2b[...]) + mb2[...], 0.0)
    x = jnp.sin(30.0 * (ts_ref[...] * w0[...] + b0[...])) * h0
    x = jnp.sin(dot(x, w1[...]) + b1[...]) * h1
    x = jnp.sin(dot(x, w2[...]) + b2[...]) * h2
    out_ref[...] = dot(x, wl[...]) + bl[...]


def _mlp(ts, emb, weights, n):
    R = 4096
    grid = n // R
    full = lambda w: pl.BlockSpec(w.shape, lambda i: (0,) * w.ndim)
    return pl.pallas_call(
        _mlp_body,
        grid=(grid,),
        in_specs=[pl.BlockSpec((R, 1), lambda i: (i, 0)),
                  pl.BlockSpec((R, EMB), lambda i: (i, 0))]
                 + [full(w) for w in weights],
        out_specs=pl.BlockSpec((R, 3), lambda i: (i, 0)),
        out_shape=jax.ShapeDtypeStruct((n, 3), jnp.float32),
    )(ts, emb, *weights)


def kernel(temporal_steps, all_coords, table_xy, table_yt, table_xt, grid3d,
           W0, b0, W1, b1, W2, b2, Wl, bl, M0, Mb0, M1, Mb1, M2, Mb2):
    b, t = temporal_steps.shape
    n = b * t
    ts = temporal_steps.reshape(n, 1)
    ac = all_coords.reshape(n, 3)
    ac0, ac1, ac2 = ac[:, 0], ac[:, 1], ac[:, 2]

    t0 = table_xy.reshape(-1)
    t1 = table_yt.reshape(-1)
    t2 = table_xt.reshape(-1)
    grid = jnp.pad(grid3d, ((0, 0), (0, 0), (0, 0), (0, C3P - C3))
                   ).reshape(TRES * XRES * YRES, C3P)

    emb = _sc_embed(ac0, ac1, ac2, t0, t1, t2, grid, n).reshape(n, EMB)

    weights = (
        M0.T, Mb0.reshape(1, HID),
        M1[:, :HID].T, M1[:, HID:].T, Mb1.reshape(1, HID),
        M2[:, :HID].T, M2[:, HID:].T, Mb2.reshape(1, HID),
        W0.T, b0.reshape(1, HID),
        W1.T, b1.reshape(1, HID),
        W2.T, b2.reshape(1, HID),
        Wl.T, bl.reshape(1, 3),
    )
    out = _mlp(ts, emb, weights, n)
    return out.reshape(b, t, 3)


# speedup vs baseline: 1.0275x; 1.0001x over previous
import functools

import numpy as np
import jax
import jax.numpy as jnp
from jax import lax
from jax.experimental import pallas as pl
from jax.experimental.pallas import tpu as pltpu
from jax.experimental.pallas import tpu_sc as plsc

L_LEVELS = 8
T_TBL = 2 ** 19
TMASK = T_TBL - 1
TRES, XRES, YRES = 32, 64, 64
C3 = 36
C3P = 48
EMB = 84
HID = 64
P2_I32 = np.int32(np.uint32(2654435761).view(np.int32))

BASE_RES = 16.0
SCALE_RES = 1.5
NC, NS = 2, 16
NW = NC * NS
B = 128

_OFF3 = [dt * (XRES * YRES) + dx * YRES + dy
         for dt in (0, 1) for dx in (0, 1) for dy in (0, 1)]
_PLANES = [(1, 2), (0, 2), (0, 1)]


def _emb_body(nblk, ac0, ac1, ac2, t0, t1, t2, grid, emb_out,
              xyz_v, idx_all, w_all, rows_all, idx3_v, w3f, rows3_v,
              dsp_v, sblock_v, sem_m, sem_s):
    wid = lax.axis_index("s") * NC + lax.axis_index("c")
    iota = lax.iota(jnp.int32, 16)
    tables = [t0, t1, t2]

    def _dsp(g, c):
        q = g * 16 + iota
        dsp_v[pl.ds(g * 16, 16)] = q * EMB
        return c
    lax.fori_loop(0, B // 16, _dsp, 0)

    def block(bi, carry):
        base = (wid * nblk + bi) * B

        for d, ac in enumerate((ac0, ac1, ac2)):
            pltpu.sync_copy(ac.at[pl.ds(base, B)], xyz_v.at[d])

        def mgrp(g, c):
            s = g * 16
            ct = xyz_v[0, pl.ds(s, 16)]
            cx = xyz_v[1, pl.ds(s, 16)]
            cy = xyz_v[2, pl.ds(s, 16)]
            pt = ct * jnp.float32(TRES - 1)
            px = cx * jnp.float32(XRES - 1)
            py = cy * jnp.float32(YRES - 1)
            ft = jnp.clip(pt.astype(jnp.int32), 0, TRES - 2)
            fx = jnp.clip(px.astype(jnp.int32), 0, XRES - 2)
            fy = jnp.clip(py.astype(jnp.int32), 0, YRES - 2)
            wt = pt - ft.astype(jnp.float32)
            wx = px - fx.astype(jnp.float32)
            wy = py - fy.astype(jnp.float32)
            fl = ft * (XRES * YRES) + fx * YRES + fy
            wts = (1.0 - wt, wt)
            wxs = (1.0 - wx, wx)
            wys = (1.0 - wy, wy)
            ci = 0
            for dt in (0, 1):
                for dx in (0, 1):
                    wtx = wts[dt] * wxs[dx]
                    for dy in (0, 1):
                        idx3_v[ci, pl.ds(s, 16)] = fl + _OFF3[ci]
                        w3f[pl.ds(ci * B + s, 16)] = wtx * wys[dy]
                        ci += 1
            return c
        lax.fori_loop(0, B // 16, mgrp, 0)

        mcps = [pltpu.async_copy(grid.at[idx3_v.at[ci]], rows3_v.at[ci],
                                 sem_m)
                for ci in range(8)]

        scps = []
        for p, (d0, d1) in enumerate(_PLANES):
            for l in range(L_LEVELS):
                resf = float(np.floor(BASE_RES * SCALE_RES ** l))
                off = l * T_TBL
                slot = p * L_LEVELS + l

                def hgrp(g, c, _d0=d0, _d1=d1, _resf=resf, _off=off,
                         _slot=slot):
                    s = g * 16
                    u = xyz_v[_d0, pl.ds(s, 16)]
                    v = xyz_v[_d1, pl.ds(s, 16)]
                    pu = u * _resf
                    pv = v * _resf
                    fu = pu.astype(jnp.int32)
                    fv = pv.astype(jnp.int32)
                    wu = pu - fu.astype(jnp.float32)
                    wv = pv - fv.astype(jnp.float32)
                    hv0 = fv * P2_I32
                    hv1 = (fv + 1) * P2_I32
                    wus = (1.0 - wu, wu)
                    wvs = (1.0 - wv, wv)
                    ci = 0
                    for du in (0, 1):
                        cu = fu + du
                        for dv in (0, 1):
                            hh = cu ^ (hv1 if dv else hv0)
                            ie = ((hh & TMASK) + _off) * 2
                            idx_all[_slot, 2 * ci, pl.ds(s, 16)] = ie
                            idx_all[_slot, 2 * ci + 1, pl.ds(s, 16)] = ie + 1
                            w_all[_slot, pl.ds(ci * B + s, 16)] = \
                                wus[du] * wvs[dv]
                            ci += 1
                    return c
                lax.fori_loop(0, B // 16, hgrp, 0)

                for k in range(8):
                    scps.append(pltpu.async_copy(
                        tables[p].at[idx_all.at[slot, k]],
                        rows_all.at[slot, pl.ds(k * B, B)],
                        sem_s))

        for cp in mcps:
            cp.wait()

        def macc(p, c):
            ws = [plsc.load_gather(w3f, [jnp.full((16,), ci * B + p, jnp.int32)])
                  for ci in range(8)]
            for gi in range(3):
                acc = ws[0] * rows3_v[0, p, pl.ds(gi * 16, 16)]
                for ci in range(1, 8):
                    acc = acc + ws[ci] * rows3_v[ci, p, pl.ds(gi * 16, 16)]
                sblock_v[pl.ds(p * EMB + 48 + gi * 16, 16)] = acc
            return c
        lax.fori_loop(0, B, macc, 0)

        for cp in scps:
            cp.wait()

        def sacc(l, c):
            p_ = l // L_LEVELS
            colbase = p_ * 16 + (l - p_ * L_LEVELS) * 2
            def agrp(g, cc):
                s = g * 16
                dd = dsp_v[pl.ds(s, 16)] + colbase
                acc0 = None
                acc1 = None
                for ci in range(4):
                    wgt = w_all[l, pl.ds(ci * B + s, 16)]
                    r0 = rows_all[l, pl.ds(2 * ci * B + s, 16)]
                    r1 = rows_all[l, pl.ds((2 * ci + 1) * B + s, 16)]
                    acc0 = wgt * r0 if acc0 is None else acc0 + wgt * r0
                    acc1 = wgt * r1 if acc1 is None else acc1 + wgt * r1
                plsc.store_scatter(sblock_v, [dd], acc0)
                plsc.store_scatter(sblock_v, [dd + 1], acc1)
                return cc
            lax.fori_loop(0, B // 16, agrp, 0)
            return c
        lax.fori_loop(0, 3 * L_LEVELS, sacc, 0)

        pltpu.sync_copy(sblock_v.at[pl.ds(0, B * EMB)],
                        emb_out.at[pl.ds(base * EMB, B * EMB)])
        return carry

    lax.fori_loop(0, nblk, block, 0)


def _sc_embed(ac0, ac1, ac2, t0, t1, t2, grid, n):
    nblk = n // (NW * B)
    mesh = plsc.VectorSubcoreMesh(core_axis_name="c", subcore_axis_name="s")
    return pl.kernel(
        functools.partial(_emb_body, nblk),
        out_type=jax.ShapeDtypeStruct((n * EMB,), jnp.float32),
        mesh=mesh,
        compiler_params=pltpu.CompilerParams(needs_layout_passes=False,
                                             use_tc_tiling_on_sc=False),
        scratch_types=[
            pltpu.VMEM((3, B), jnp.float32),
            pltpu.VMEM((24, 8, B), jnp.int32),
            pltpu.VMEM((24, 4 * B), jnp.float32),
            pltpu.VMEM((24, 8 * B), jnp.float32),
            pltpu.VMEM((8, B), jnp.int32),
            pltpu.VMEM((8 * B,), jnp.float32),
            pltpu.VMEM((8, B, C3P), jnp.float32),
            pltpu.VMEM((B,), jnp.int32),
            pltpu.VMEM((B * EMB + 16,), jnp.float32),
            pltpu.SemaphoreType.DMA,
            pltpu.SemaphoreType.DMA,
        ],
    )(ac0, ac1, ac2, t0, t1, t2, grid)


def _mlp_body(ts_ref, emb_ref, m0, mb0, m1a, m1b, mb1, m2a, m2b, mb2,
              w0, b0, w1, b1, w2, b2, wl, bl, out_ref):
    dot = functools.partial(jnp.dot, precision=lax.Precision.HIGHEST)
    emb = emb_ref[...]
    h0 = jnp.maximum(dot(emb, m0[...]) + mb0[...], 0.0)
    h1 = jnp.maximum(dot(h0, m1a[...]) + dot(emb, m1b[...]) + mb1[...], 0.0)
    h2 = jnp.maximum(dot(h1, m2a[...]) + dot(emb, m2b[...]) + mb2[...], 0.0)
    x = jnp.sin(30.0 * (ts_ref[...] * w0[...] + b0[...])) * h0
    x = jnp.sin(dot(x, w1[...]) + b1[...]) * h1
    x = jnp.sin(dot(x, w2[...]) + b2[...]) * h2
    out_ref[...] = dot(x, wl[...]) + bl[...]


def _mlp(ts, emb, weights, n):
    R = 4096
    grid = n // R
    full = lambda w: pl.BlockSpec(w.shape, lambda i: (0,) * w.ndim)
    return pl.pallas_call(
        _mlp_body,
        grid=(grid,),
        in_specs=[pl.BlockSpec((R, 1), lambda i: (i, 0)),
                  pl.BlockSpec((R, EMB), lambda i: (i, 0))]
                 + [full(w) for w in weights],
        out_specs=pl.BlockSpec((R, 3), lambda i: (i, 0)),
        out_shape=jax.ShapeDtypeStruct((n, 3), jnp.float32),
    )(ts, emb, *weights)


def kernel(temporal_steps, all_coords, table_xy, table_yt, table_xt, grid3d,
           W0, b0, W1, b1, W2, b2, Wl, bl, M0, Mb0, M1, Mb1, M2, Mb2):
    b, t = temporal_steps.shape
    n = b * t
    ts = temporal_steps.reshape(n, 1)
    ac = all_coords.reshape(n, 3)
    ac0, ac1, ac2 = ac[:, 0], ac[:, 1], ac[:, 2]

    t0 = table_xy.reshape(-1)
    t1 = table_yt.reshape(-1)
    t2 = table_xt.reshape(-1)
    grid = jnp.pad(grid3d, ((0, 0), (0, 0), (0, 0), (0, C3P - C3))
                   ).reshape(TRES * XRES * YRES, C3P)

    emb = _sc_embed(ac0, ac1, ac2, t0, t1, t2, grid, n).reshape(n, EMB)

    weights = (
        M0.T, Mb0.reshape(1, HID),
        M1[:, :HID].T, M1[:, HID:].T, Mb1.reshape(1, HID),
        M2[:, :HID].T, M2[:, HID:].T, Mb2.reshape(1, HID),
        W0.T, b0.reshape(1, HID),
        W1.T, b1.reshape(1, HID),
        W2.T, b2.reshape(1, HID),
        Wl.T, bl.reshape(1, 3),
    )
    out = _mlp(ts, emb, weights, n)
    return out.reshape(b, t, 3)
